# baseline (speedup 1.0000x reference)
import jax
import jax.numpy as jnp
from jax import lax
from jax.experimental import pallas as pl
from jax.experimental.pallas import tpu as pltpu


def _body(x_ref, w1_ref, b1_ref, w2_ref, b2_ref,
          wl1_ref, bl1_ref, wl2_ref, bl2_ref, wl3_ref, bl3_ref,
          o_ref, out1, mra, mrb, m2a, m2b, feat):

    def c1_body(i, _):
        c = i // 39
        j = i % 39
        for s in range(5):
            h = 5 * j + s
            ra = None
            rb = None
            for dy in range(5):
                for dx in range(5):
                    tap = x_ref[h + dy, dx:dx + 195, :]
                    ta = w1_ref[c, dy * 5 + dx] * tap
                    tb = w1_ref[c + 2, dy * 5 + dx] * tap
                    ra = ta if ra is None else ra + ta
                    rb = tb if rb is None else rb + tb
            if s == 0:
                mra[...] = ra
                mrb[...] = rb
            else:
                mra[...] = jnp.maximum(mra[...], ra)
                mrb[...] = jnp.maximum(mrb[...], rb)
        cma = None
        cmb = None
        for s5 in range(5):
            sla = mra[pl.ds(s5, 39, 5), :]
            slb = mrb[pl.ds(s5, 39, 5), :]
            cma = sla if cma is None else jnp.maximum(cma, sla)
            cmb = slb if cmb is None else jnp.maximum(cmb, slb)
        out1[i] = jnp.maximum(cma + b1_ref[c], 0.0)
        out1[i + 78] = jnp.maximum(cmb + b1_ref[c + 2], 0.0)
        return 0

    lax.fori_loop(0, 2 * 39, c1_body, 0)

    feat[...] = jnp.zeros((448, 128), jnp.float32)

    def c2_body(i, _):
        co = i // 7
        j = i % 7
        for s in range(5):
            ra = None
            rb = None
            for ci in range(4):
                for dy in range(5):
                    row = ci * 39 + 5 * j + s + dy
                    for dx in range(5):
                        tap = out1[row, dx:dx + 35, :]
                        k = (ci * 5 + dy) * 5 + dx
                        ta = w2_ref[co, k] * tap
                        tb = w2_ref[co + 4, k] * tap
                        ra = ta if ra is None else ra + ta
                        rb = tb if rb is None else rb + tb
            if s == 0:
                m2a[...] = ra
                m2b[...] = rb
            else:
                m2a[...] = jnp.maximum(m2a[...], ra)
                m2b[...] = jnp.maximum(m2b[...], rb)
        cma = None
        cmb = None
        for s5 in range(5):
            sla = m2a[pl.ds(s5, 7, 5), :]
            slb = m2b[pl.ds(s5, 7, 5), :]
            cma = sla if cma is None else jnp.maximum(cma, sla)
            cmb = slb if cmb is None else jnp.maximum(cmb, slb)
        base = pl.multiple_of(8 * i, 8)
        feat[pl.ds(base, 7), :] = jnp.maximum(cma + b2_ref[co], 0.0)
        base2 = pl.multiple_of(8 * i + 224, 8)
        feat[pl.ds(base2, 7), :] = jnp.maximum(cmb + b2_ref[co + 4], 0.0)
        return 0

    lax.fori_loop(0, 4 * 7, c2_body, 0)

    f = feat[...]
    h1 = lax.dot_general(f, wl1_ref[...], (((0,), (0,)), ((), ())),
                         preferred_element_type=jnp.float32)
    h1 = jnp.maximum(h1 + bl1_ref[...], 0.0)
    h2 = jnp.dot(h1, wl2_ref[...], preferred_element_type=jnp.float32)
    h2 = jnp.maximum(h2 + bl2_ref[...], 0.0)
    h3 = jnp.dot(h2, wl3_ref[...], preferred_element_type=jnp.float32)
    o_ref[...] = jnp.maximum(h3 + bl3_ref[...], 0.0)


def kernel(x, w1, b1, w2, b2, wl1, bl1, wl2, bl2, wl3, bl3):
    B = x.shape[0]
    assert B % 128 == 0
    nb = B // 128

    xt = jnp.transpose(x.reshape(B, 200, 200), (1, 2, 0))
    w1f = w1.reshape(4, 25)
    w2f = w2.reshape(8, 100)

    fi = jnp.arange(392)
    rows = (fi // 49) * 56 + ((fi // 7) % 7) * 8 + (fi % 7)
    wl1p = jnp.zeros((448, 500), jnp.float32).at[rows].set(wl1)

    full = lambda shape: pl.BlockSpec(shape, lambda i: (0,) * len(shape))
    smem = pl.BlockSpec(memory_space=pltpu.MemorySpace.SMEM)

    return pl.pallas_call(
        _body,
        out_shape=jax.ShapeDtypeStruct((B, 361), jnp.float32),
        grid=(nb,),
        in_specs=[
            pl.BlockSpec((200, 200, 128), lambda i: (0, 0, i)),
            smem,
            smem,
            smem,
            smem,
            full((448, 500)),
            full((1, 500)),
            full((500, 400)),
            full((1, 400)),
            full((400, 361)),
            full((1, 361)),
        ],
        out_specs=pl.BlockSpec((128, 361), lambda i: (i, 0)),
        scratch_shapes=[
            pltpu.VMEM((156, 39, 128), jnp.float32),
            pltpu.VMEM((195, 128), jnp.float32),
            pltpu.VMEM((195, 128), jnp.float32),
            pltpu.VMEM((35, 128), jnp.float32),
            pltpu.VMEM((35, 128), jnp.float32),
            pltpu.VMEM((448, 128), jnp.float32),
        ],
        compiler_params=pltpu.CompilerParams(
            dimension_semantics=("parallel",)),
    )(xt, w1f, b1, w2f, b2, wl1p, bl1.reshape(1, 500),
      wl2, bl2.reshape(1, 400), wl3, bl3.reshape(1, 361))


# speedup vs baseline: 1.1019x; 1.1019x over previous
import jax
import jax.numpy as jnp
from jax import lax
from jax.experimental import pallas as pl
from jax.experimental.pallas import tpu as pltpu


def _body(x_ref, w1_ref, b1_ref, w2_ref, b2_ref,
          wl1_ref, bl1_ref, wl2_ref, bl2_ref, wl3_ref, bl3_ref,
          o_ref, out1s, mrow, m2, feat):

    def c1_body(i, _):
        c = i // 39
        j = i % 39
        m = None
        for s in range(5):
            h = 5 * j + s
            r = None
            for dy in range(5):
                for dx in range(5):
                    tap = w1_ref[c, dy * 5 + dx] * x_ref[h + dy, dx:dx + 195, :]
                    r = tap if r is None else r + tap
            m = r if m is None else jnp.maximum(m, r)
        mrow[...] = m
        cm = None
        for s5 in range(5):
            sl = mrow[pl.ds(s5, 39, 5), :]
            cm = sl if cm is None else jnp.maximum(cm, sl)
        v = jnp.maximum(cm + b1_ref[c], 0.0)
        for dx in range(5):
            out1s[dx, i] = v[dx:dx + 35, :]
        return 0

    lax.fori_loop(0, 4 * 39, c1_body, 0)

    feat[...] = jnp.zeros((448, 128), jnp.float32)

    def c2_body(i, _):
        co = i // 7
        j = i % 7
        m = None
        for s in range(5):
            r = None
            for ci in range(4):
                for dy in range(5):
                    row = ci * 39 + 5 * j + s + dy
                    for dx in range(5):
                        tap = (w2_ref[co, (ci * 5 + dy) * 5 + dx]
                               * out1s[dx, row])
                        r = tap if r is None else r + tap
            m = r if m is None else jnp.maximum(m, r)
        m2[...] = m
        cm = None
        for s5 in range(5):
            sl = m2[pl.ds(s5, 7, 5), :]
            cm = sl if cm is None else jnp.maximum(cm, sl)
        base = pl.multiple_of(8 * i, 8)
        feat[pl.ds(base, 7), :] = jnp.maximum(cm + b2_ref[co], 0.0)
        return 0

    lax.fori_loop(0, 8 * 7, c2_body, 0)

    f = feat[...]
    h1 = lax.dot_general(f, wl1_ref[...], (((0,), (0,)), ((), ())),
                         preferred_element_type=jnp.float32)
    h1 = jnp.maximum(h1 + bl1_ref[...], 0.0)
    h2 = jnp.dot(h1, wl2_ref[...], preferred_element_type=jnp.float32)
    h2 = jnp.maximum(h2 + bl2_ref[...], 0.0)
    h3 = jnp.dot(h2, wl3_ref[...], preferred_element_type=jnp.float32)
    o_ref[...] = jnp.maximum(h3 + bl3_ref[...], 0.0)


def kernel(x, w1, b1, w2, b2, wl1, bl1, wl2, bl2, wl3, bl3):
    B = x.shape[0]
    assert B % 128 == 0
    nb = B // 128

    xt = jnp.transpose(x.reshape(B, 200, 200), (1, 2, 0))
    w1f = w1.reshape(4, 25)
    w2f = w2.reshape(8, 100)

    fi = jnp.arange(392)
    rows = (fi // 49) * 56 + ((fi // 7) % 7) * 8 + (fi % 7)
    wl1p = jnp.zeros((448, 500), jnp.float32).at[rows].set(wl1)

    full = lambda shape: pl.BlockSpec(shape, lambda i: (0,) * len(shape))
    smem = pl.BlockSpec(memory_space=pltpu.MemorySpace.SMEM)

    return pl.pallas_call(
        _body,
        out_shape=jax.ShapeDtypeStruct((B, 361), jnp.float32),
        grid=(nb,),
        in_specs=[
            pl.BlockSpec((200, 200, 128), lambda i: (0, 0, i)),
            smem,
            smem,
            smem,
            smem,
            full((448, 500)),
            full((1, 500)),
            full((500, 400)),
            full((1, 400)),
            full((400, 361)),
            full((1, 361)),
        ],
        out_specs=pl.BlockSpec((128, 361), lambda i: (i, 0)),
        scratch_shapes=[
            pltpu.VMEM((5, 156, 35, 128), jnp.float32),
            pltpu.VMEM((195, 128), jnp.float32),
            pltpu.VMEM((35, 128), jnp.float32),
            pltpu.VMEM((448, 128), jnp.float32),
        ],
        compiler_params=pltpu.CompilerParams(
            dimension_semantics=("parallel",)),
    )(xt, w1f, b1, w2f, b2, wl1p, bl1.reshape(1, 500),
      wl2, bl2.reshape(1, 400), wl3, bl3.reshape(1, 361))
